# baseline (speedup 1.0000x reference)
import functools

import jax
import jax.numpy as jnp
from jax import lax
from jax.experimental import pallas as pl
from jax.experimental.pallas import tpu as pltpu
from jax.experimental.pallas import tpu_sc as plsc

B, N, K, DIM, HID = 2, 4096, 16, 256, 512
DIM2 = 2 * DIM
BN = 256
NB = N // BN
BM = 1024
CH = 128
NSC, NSUB = 2, 16
NW = NSC * NSUB
PH = 1
NH = N // PH
NPW = (K * NH) // NW
CPW = NPW // CH


def _fold_body(wq, wa1, wkv, wp2, wqa_ref, wg_ref, wp2cat_ref):
    a1 = wa1[...]
    wqa_ref[...] = jnp.dot(wq[...], a1, preferred_element_type=jnp.float32)
    wg_ref[:, :DIM] = jnp.dot(wkv[:, :DIM], a1, preferred_element_type=jnp.float32)
    wg_ref[:, DIM:] = wkv[:, DIM:]
    wp2cat_ref[:, :DIM] = wp2[...]
    wp2cat_ref[:, DIM:] = jnp.dot(wp2[...], a1, preferred_element_type=jnp.float32)


def _rne16(x):
    u = lax.bitcast_convert_type(x, jnp.int32)
    return (u + 0x7FFF + ((u >> 16) & 1)) >> 16


def _pre_body(feats, g1b1, wqa, wg, qa_ref, g_ref):
    x = feats[...]
    m = jnp.mean(x, axis=-1, keepdims=True)
    xc = x - m
    v = jnp.mean(xc * xc, axis=-1, keepdims=True)
    xn = xc * lax.rsqrt(v + 1e-5) * g1b1[0:1, :] + g1b1[1:2, :]
    qa_ref[...] = jnp.dot(xn, wqa[...], preferred_element_type=jnp.float32)
    kav = jnp.dot(xn, wg[...], preferred_element_type=jnp.float32)
    ka_b = _rne16(kav[:, :DIM]) & 0xFFFF
    v_b = _rne16(kav[:, DIM:])
    g_ref[...] = ka_b | (v_b << 16)


def _gather_body(gtab, xyzt, idxf, gg_out, xg_out, idx_v, xyz_v,
                 gbuf0, gbuf1, xbuf0, xbuf1,
                 sem_g0, sem_g1, sem_w0, sem_w1, sem_x0, sem_x1):
    wid = lax.axis_index("s") * NSC + lax.axis_index("c")
    k_ = wid // 2
    sub = wid & 1
    pltpu.sync_copy(idxf.at[wid], idx_v)
    pltpu.sync_copy(xyzt, xyz_v)
    gbufs = (gbuf0, gbuf1)
    xbufs = (xbuf0, xbuf1)
    gsems = (sem_g0, sem_g1)
    wsems = (sem_w0, sem_w1)
    xsems = (sem_x0, sem_x1)
    lane = lax.iota(jnp.int32, 16)
    nsub = lane >> 2
    comp = lane & 3
    nbase = sub * NPW

    def start_gather(c):
        isl = idx_v.at[pl.ds(c * CH, CH)]
        return pltpu.async_copy(gtab.at[isl], gbufs[c % 2], gsems[c % 2])

    gathers = {0: start_gather(0)}
    writes = {}
    xwrites = {}
    for c in range(CPW):
        b = c % 2
        if c + 1 < CPW:
            if c >= 1:
                writes[c - 1].wait()
            gathers[c + 1] = start_gather(c + 1)
        if c >= 2:
            xwrites[c - 2].wait()
        xb = xbufs[b]
        for mq in range(CH // 4):
            iv = plsc.load_gather(idx_v, [c * CH + mq * 4 + nsub])
            g = plsc.load_gather(xyz_v, [iv * 4 + comp])
            plsc.store_scatter(xb, [comp, mq * 4 + nsub], g)
        xwrites[c] = pltpu.async_copy(
            xb, xg_out.at[k_, :, pl.ds(nbase + c * CH, CH)], xsems[b])
        gathers[c].wait()
        writes[c] = pltpu.async_copy(gbufs[b], gg_out.at[wid, pl.ds(c * CH, CH)],
                                     wsems[b])
    writes[CPW - 2].wait()
    writes[CPW - 1].wait()
    xwrites[CPW - 2].wait()
    xwrites[CPW - 1].wait()


def _attn_body(feats, xpcm, qa, gg, xg, wp1cm, wp2cat, wa2, wproj, wf1, wf2,
               g2b2, out_ref):
    bf = jnp.bfloat16
    f = feats[...]
    xp = xpcm[...]
    q = qa[...]
    w1b = wp1cm[...].astype(bf)
    w2b = wp2cat[...].astype(bf)
    wab = wa2[...].astype(bf)
    m = jnp.full((BN, DIM), -1e30, jnp.float32)
    s = jnp.zeros((BN, DIM), jnp.float32)
    acc = jnp.zeros((BN, DIM), jnp.float32)
    for k in range(K):
        kav = gg[0, k]
        ka = lax.bitcast_convert_type(kav << 16, jnp.float32)
        v = lax.bitcast_convert_type(kav & jnp.int32(-65536), jnp.float32)
        relcm = xg[0, k] - xp
        h = jnp.maximum(
            lax.dot_general(relcm.astype(bf), w1b, (((0,), (0,)), ((), ())),
                            preferred_element_type=jnp.float32), 0.0)
        pp = jnp.dot(h.astype(bf), w2b, preferred_element_type=jnp.float32)
        t = jnp.dot(jnp.maximum(q - ka + pp[:, DIM:], 0.0).astype(bf), wab,
                    preferred_element_type=jnp.float32)
        mn = jnp.maximum(m, t)
        sc = jnp.exp(m - mn)
        e = jnp.exp(t - mn)
        s = s * sc + e
        acc = acc * sc + e * (v + pp[:, :DIM])
        m = mn
    out = acc / s
    y = f + jnp.dot(out.astype(bf), wproj[...].astype(bf),
                    preferred_element_type=jnp.float32)
    mu = jnp.mean(y, axis=-1, keepdims=True)
    yc = y - mu
    var = jnp.mean(yc * yc, axis=-1, keepdims=True)
    ln = yc * lax.rsqrt(var + 1e-5) * g2b2[0:1, :] + g2b2[1:2, :]
    z = y + jnp.dot(
        jnp.maximum(jnp.dot(ln.astype(bf), wf1[...].astype(bf),
                            preferred_element_type=jnp.float32), 0.0).astype(bf),
        wf2[...].astype(bf), preferred_element_type=jnp.float32)
    out_ref[...] = z


def _fold_call(wq, wa1, wkv, wp2):
    return pl.pallas_call(
        _fold_body,
        out_shape=(
            jax.ShapeDtypeStruct((DIM, DIM), jnp.float32),
            jax.ShapeDtypeStruct((DIM, DIM2), jnp.float32),
            jax.ShapeDtypeStruct((DIM, DIM2), jnp.float32),
        ),
    )(wq, wa1, wkv, wp2)


def _pre_call(feats2, g1b1, wqa, wg):
    nblk = (B * N) // BM
    return pl.pallas_call(
        _pre_body,
        grid=(nblk,),
        in_specs=[
            pl.BlockSpec((BM, DIM), lambda i: (i, 0)),
            pl.BlockSpec((2, DIM), lambda i: (0, 0)),
            pl.BlockSpec((DIM, DIM), lambda i: (0, 0)),
            pl.BlockSpec((DIM, DIM2), lambda i: (0, 0)),
        ],
        out_specs=(
            pl.BlockSpec((BM, DIM), lambda i: (i, 0)),
            pl.BlockSpec((BM, DIM), lambda i: (i, 0)),
        ),
        out_shape=(
            jax.ShapeDtypeStruct((B * N, DIM), jnp.float32),
            jax.ShapeDtypeStruct((B * N, DIM), jnp.int32),
        ),
    )(feats2, g1b1, wqa, wg)


def _gather_call(gtab, xyzt, idxf_p):
    k = functools.partial(
        pl.kernel,
        mesh=plsc.VectorSubcoreMesh(core_axis_name="c", subcore_axis_name="s"),
        out_type=[
            jax.ShapeDtypeStruct((NW, NPW, DIM), jnp.int32),
            jax.ShapeDtypeStruct((K, 4, NH), jnp.float32),
        ],
        scratch_types=[
            pltpu.VMEM((NPW,), jnp.int32),
            pltpu.VMEM((B * N * 4,), jnp.float32),
            pltpu.VMEM((CH, DIM), jnp.int32),
            pltpu.VMEM((CH, DIM), jnp.int32),
            pltpu.VMEM((4, CH), jnp.float32),
            pltpu.VMEM((4, CH), jnp.float32),
            pltpu.SemaphoreType.DMA,
            pltpu.SemaphoreType.DMA,
            pltpu.SemaphoreType.DMA,
            pltpu.SemaphoreType.DMA,
            pltpu.SemaphoreType.DMA,
            pltpu.SemaphoreType.DMA,
        ],
        compiler_params=pltpu.CompilerParams(needs_layout_passes=False),
    )(_gather_body)
    return k(gtab, xyzt, idxf_p)


def _attn_call(base, feats2, xpcm, qa, gg_p, xg_p, wp1cm, wp2cat, wa2, wproj,
               wf1, wf2, g2b2):
    nbh = NH // BN
    row = lambda i, base=base: (base + i, 0)
    return pl.pallas_call(
        _attn_body,
        grid=(nbh,),
        in_specs=[
            pl.BlockSpec((BN, DIM), row),
            pl.BlockSpec((4, BN), lambda i, base=base: (0, base + i)),
            pl.BlockSpec((BN, DIM), row),
            pl.BlockSpec((1, K, BN, DIM), lambda i: (0, 0, i, 0)),
            pl.BlockSpec((1, K, 4, BN), lambda i: (0, 0, 0, i)),
            pl.BlockSpec((4, DIM), lambda i: (0, 0)),
            pl.BlockSpec((DIM, DIM2), lambda i: (0, 0)),
            pl.BlockSpec((DIM, DIM), lambda i: (0, 0)),
            pl.BlockSpec((DIM, DIM), lambda i: (0, 0)),
            pl.BlockSpec((DIM, HID), lambda i: (0, 0)),
            pl.BlockSpec((HID, DIM), lambda i: (0, 0)),
            pl.BlockSpec((2, DIM), lambda i: (0, 0)),
        ],
        out_specs=pl.BlockSpec((BN, DIM), lambda i: (i, 0)),
        out_shape=jax.ShapeDtypeStruct((NH, DIM), jnp.float32),
    )(feats2, xpcm, qa, gg_p, xg_p, wp1cm, wp2cat, wa2, wproj, wf1, wf2, g2b2)


def kernel(xyz, feats, idx, g1, b1, g2, b2, Wq, Wkv, Wp1, Wp2, Wa1, Wa2, Wproj, Wf1, Wf2):
    feats2 = feats.reshape(B * N, DIM)
    xyz4 = jnp.pad(xyz, ((0, 0), (0, 0), (0, 1))).reshape(B * N, 4)
    xyzt = xyz4.reshape(B * N * 4)
    xpcm = xyz4.T
    wp1cm = jnp.pad(Wp1, ((0, 1), (0, 0)))
    g1b1 = jnp.stack([g1, b1])
    g2b2 = jnp.stack([g2, b2])
    idxf = (idx + (jnp.arange(B, dtype=jnp.int32) * N)[:, None, None]
            ).transpose(0, 2, 1).reshape(B, K, PH, NH).transpose(0, 2, 1, 3
            ).reshape(B, PH, NW, NPW)

    wqa, wg, wp2cat = _fold_call(Wq, Wa1, Wkv, Wp2)
    qa, gtab = _pre_call(feats2, g1b1, wqa, wg)
    zs = []
    for b in range(B):
        for h in range(PH):
            gg_p, xg_p = _gather_call(gtab, xyzt, idxf[b, h])
            base = b * NB + h * (NH // BN)
            zs.append(_attn_call(base, feats2, xpcm, qa,
                                 gg_p.reshape(1, K, NH, DIM),
                                 xg_p.reshape(1, K, 4, NH),
                                 wp1cm, wp2cat, Wa2, Wproj, Wf1, Wf2, g2b2))
    return jnp.concatenate(zs).reshape(B, N, DIM)


# speedup vs baseline: 1.0669x; 1.0669x over previous
import functools

import jax
import jax.numpy as jnp
from jax import lax
from jax.experimental import pallas as pl
from jax.experimental.pallas import tpu as pltpu
from jax.experimental.pallas import tpu_sc as plsc

B, N, K, DIM, HID = 2, 4096, 16, 256, 512
DIM2 = 2 * DIM
BN = 256
NB = N // BN
BM = 1024
CH = 128
NSC, NSUB = 2, 16
NW = NSC * NSUB
PH = 2
NH = N // PH
NPW = (K * NH) // NW
CPW = NPW // CH


def _fold_body(wq, wa1, wkv, wp2, wqa_ref, wg_ref, wp2cat_ref):
    a1 = wa1[...]
    wqa_ref[...] = jnp.dot(wq[...], a1, preferred_element_type=jnp.float32)
    wg_ref[:, :DIM] = jnp.dot(wkv[:, :DIM], a1, preferred_element_type=jnp.float32)
    wg_ref[:, DIM:] = wkv[:, DIM:]
    wp2cat_ref[:, :DIM] = wp2[...]
    wp2cat_ref[:, DIM:] = jnp.dot(wp2[...], a1, preferred_element_type=jnp.float32)


def _rne16(x):
    u = lax.bitcast_convert_type(x, jnp.int32)
    return (u + 0x7FFF + ((u >> 16) & 1)) >> 16


def _pre_body(feats, g1b1, wqa, wg, qa_ref, g_ref):
    x = feats[...]
    m = jnp.mean(x, axis=-1, keepdims=True)
    xc = x - m
    v = jnp.mean(xc * xc, axis=-1, keepdims=True)
    xn = xc * lax.rsqrt(v + 1e-5) * g1b1[0:1, :] + g1b1[1:2, :]
    qa_ref[...] = jnp.dot(xn, wqa[...], preferred_element_type=jnp.float32)
    kav = jnp.dot(xn, wg[...], preferred_element_type=jnp.float32)
    ka_b = _rne16(kav[:, :DIM]) & 0xFFFF
    v_b = _rne16(kav[:, DIM:])
    g_ref[...] = ka_b | (v_b << 16)


def _gather_body(gtab, xyzt, idxf, gg_out, xg_out, idx_v, xyz_v,
                 gbuf0, gbuf1, xbuf0, xbuf1,
                 sem_g0, sem_g1, sem_w0, sem_w1, sem_x0, sem_x1):
    wid = lax.axis_index("s") * NSC + lax.axis_index("c")
    k_ = wid // 2
    sub = wid & 1
    pltpu.sync_copy(idxf.at[wid], idx_v)
    pltpu.sync_copy(xyzt, xyz_v)
    gbufs = (gbuf0, gbuf1)
    xbufs = (xbuf0, xbuf1)
    gsems = (sem_g0, sem_g1)
    wsems = (sem_w0, sem_w1)
    xsems = (sem_x0, sem_x1)
    lane = lax.iota(jnp.int32, 16)
    nsub = lane >> 2
    comp = lane & 3
    nbase = sub * NPW

    def start_gather(c):
        isl = idx_v.at[pl.ds(c * CH, CH)]
        return pltpu.async_copy(gtab.at[isl], gbufs[c % 2], gsems[c % 2])

    gathers = {0: start_gather(0)}
    writes = {}
    xwrites = {}
    for c in range(CPW):
        b = c % 2
        if c + 1 < CPW:
            if c >= 1:
                writes[c - 1].wait()
            gathers[c + 1] = start_gather(c + 1)
        if c >= 2:
            xwrites[c - 2].wait()
        xb = xbufs[b]
        for mq in range(CH // 4):
            iv = plsc.load_gather(idx_v, [c * CH + mq * 4 + nsub])
            g = plsc.load_gather(xyz_v, [iv * 4 + comp])
            plsc.store_scatter(xb, [comp, mq * 4 + nsub], g)
        xwrites[c] = pltpu.async_copy(
            xb, xg_out.at[k_, :, pl.ds(nbase + c * CH, CH)], xsems[b])
        gathers[c].wait()
        writes[c] = pltpu.async_copy(gbufs[b], gg_out.at[wid, pl.ds(c * CH, CH)],
                                     wsems[b])
    writes[CPW - 2].wait()
    writes[CPW - 1].wait()
    xwrites[CPW - 2].wait()
    xwrites[CPW - 1].wait()


def _attn_body(feats, xpcm, qa, gg, xg, wp1cm, wp2cat, wa2, wproj, wf1, wf2,
               g2b2, out_ref):
    bf = jnp.bfloat16
    f = feats[...]
    xp = xpcm[...]
    q = qa[...]
    w1b = wp1cm[...].astype(bf)
    w2b = wp2cat[...].astype(bf)
    wab = wa2[...].astype(bf)
    m = jnp.full((BN, DIM), -1e30, jnp.float32)
    s = jnp.zeros((BN, DIM), jnp.float32)
    acc = jnp.zeros((BN, DIM), jnp.float32)
    for k in range(K):
        kav = gg[0, k]
        ka = lax.bitcast_convert_type(kav << 16, jnp.float32)
        v = lax.bitcast_convert_type(kav & jnp.int32(-65536), jnp.float32)
        relcm = xg[0, k] - xp
        h = jnp.maximum(
            lax.dot_general(relcm.astype(bf), w1b, (((0,), (0,)), ((), ())),
                            preferred_element_type=jnp.float32), 0.0)
        pp = jnp.dot(h.astype(bf), w2b, preferred_element_type=jnp.float32)
        t = jnp.dot(jnp.maximum(q - ka + pp[:, DIM:], 0.0).astype(bf), wab,
                    preferred_element_type=jnp.float32)
        mn = jnp.maximum(m, t)
        sc = jnp.exp(m - mn)
        e = jnp.exp(t - mn)
        s = s * sc + e
        acc = acc * sc + e * (v + pp[:, :DIM])
        m = mn
    out = acc / s
    y = f + jnp.dot(out.astype(bf), wproj[...].astype(bf),
                    preferred_element_type=jnp.float32)
    mu = jnp.mean(y, axis=-1, keepdims=True)
    yc = y - mu
    var = jnp.mean(yc * yc, axis=-1, keepdims=True)
    ln = yc * lax.rsqrt(var + 1e-5) * g2b2[0:1, :] + g2b2[1:2, :]
    z = y + jnp.dot(
        jnp.maximum(jnp.dot(ln.astype(bf), wf1[...].astype(bf),
                            preferred_element_type=jnp.float32), 0.0).astype(bf),
        wf2[...].astype(bf), preferred_element_type=jnp.float32)
    out_ref[...] = z


def _fold_call(wq, wa1, wkv, wp2):
    return pl.pallas_call(
        _fold_body,
        out_shape=(
            jax.ShapeDtypeStruct((DIM, DIM), jnp.float32),
            jax.ShapeDtypeStruct((DIM, DIM2), jnp.float32),
            jax.ShapeDtypeStruct((DIM, DIM2), jnp.float32),
        ),
    )(wq, wa1, wkv, wp2)


def _pre_call(feats2, g1b1, wqa, wg):
    nblk = (B * N) // BM
    return pl.pallas_call(
        _pre_body,
        grid=(nblk,),
        in_specs=[
            pl.BlockSpec((BM, DIM), lambda i: (i, 0)),
            pl.BlockSpec((2, DIM), lambda i: (0, 0)),
            pl.BlockSpec((DIM, DIM), lambda i: (0, 0)),
            pl.BlockSpec((DIM, DIM2), lambda i: (0, 0)),
        ],
        out_specs=(
            pl.BlockSpec((BM, DIM), lambda i: (i, 0)),
            pl.BlockSpec((BM, DIM), lambda i: (i, 0)),
        ),
        out_shape=(
            jax.ShapeDtypeStruct((B * N, DIM), jnp.float32),
            jax.ShapeDtypeStruct((B * N, DIM), jnp.int32),
        ),
    )(feats2, g1b1, wqa, wg)


def _gather_call(gtab, xyzt, idxf_p):
    k = functools.partial(
        pl.kernel,
        mesh=plsc.VectorSubcoreMesh(core_axis_name="c", subcore_axis_name="s"),
        out_type=[
            jax.ShapeDtypeStruct((NW, NPW, DIM), jnp.int32),
            jax.ShapeDtypeStruct((K, 4, NH), jnp.float32),
        ],
        scratch_types=[
            pltpu.VMEM((NPW,), jnp.int32),
            pltpu.VMEM((N * 4,), jnp.float32),
            pltpu.VMEM((CH, DIM), jnp.int32),
            pltpu.VMEM((CH, DIM), jnp.int32),
            pltpu.VMEM((4, CH), jnp.float32),
            pltpu.VMEM((4, CH), jnp.float32),
            pltpu.SemaphoreType.DMA,
            pltpu.SemaphoreType.DMA,
            pltpu.SemaphoreType.DMA,
            pltpu.SemaphoreType.DMA,
            pltpu.SemaphoreType.DMA,
            pltpu.SemaphoreType.DMA,
        ],
        compiler_params=pltpu.CompilerParams(needs_layout_passes=False),
    )(_gather_body)
    return k(gtab, xyzt, idxf_p)


def _attn_call(base, feats2, xpcm, qa, gg_p, xg_p, wp1cm, wp2cat, wa2, wproj,
               wf1, wf2, g2b2):
    nbh = NH // BN
    row = lambda i, base=base: (base + i, 0)
    return pl.pallas_call(
        _attn_body,
        grid=(nbh,),
        in_specs=[
            pl.BlockSpec((BN, DIM), row),
            pl.BlockSpec((4, BN), lambda i, base=base: (0, base + i)),
            pl.BlockSpec((BN, DIM), row),
            pl.BlockSpec((1, K, BN, DIM), lambda i: (0, 0, i, 0)),
            pl.BlockSpec((1, K, 4, BN), lambda i: (0, 0, 0, i)),
            pl.BlockSpec((4, DIM), lambda i: (0, 0)),
            pl.BlockSpec((DIM, DIM2), lambda i: (0, 0)),
            pl.BlockSpec((DIM, DIM), lambda i: (0, 0)),
            pl.BlockSpec((DIM, DIM), lambda i: (0, 0)),
            pl.BlockSpec((DIM, HID), lambda i: (0, 0)),
            pl.BlockSpec((HID, DIM), lambda i: (0, 0)),
            pl.BlockSpec((2, DIM), lambda i: (0, 0)),
        ],
        out_specs=pl.BlockSpec((BN, DIM), lambda i: (i, 0)),
        out_shape=jax.ShapeDtypeStruct((NH, DIM), jnp.float32),
    )(feats2, xpcm, qa, gg_p, xg_p, wp1cm, wp2cat, wa2, wproj, wf1, wf2, g2b2)


def kernel(xyz, feats, idx, g1, b1, g2, b2, Wq, Wkv, Wp1, Wp2, Wa1, Wa2, Wproj, Wf1, Wf2):
    feats2 = feats.reshape(B * N, DIM)
    xyz4 = jnp.pad(xyz, ((0, 0), (0, 0), (0, 1))).reshape(B * N, 4)
    xpcm = xyz4.T
    wp1cm = jnp.pad(Wp1, ((0, 1), (0, 0)))
    g1b1 = jnp.stack([g1, b1])
    g2b2 = jnp.stack([g2, b2])
    idxf = idx.transpose(0, 2, 1).reshape(B, K, PH, NH).transpose(0, 2, 1, 3
           ).reshape(B, PH, NW, NPW)

    wqa, wg, wp2cat = _fold_call(Wq, Wa1, Wkv, Wp2)
    qa, gtab = _pre_call(feats2, g1b1, wqa, wg)
    zs = []
    for b in range(B):
        gtab_b = gtab[b * N:(b + 1) * N]
        xyzt_b = xyz4[b * N:(b + 1) * N].reshape(N * 4)
        for h in range(PH):
            gg_p, xg_p = _gather_call(gtab_b, xyzt_b, idxf[b, h])
            base = b * NB + h * (NH // BN)
            zs.append(_attn_call(base, feats2, xpcm, qa,
                                 gg_p.reshape(1, K, NH, DIM),
                                 xg_p.reshape(1, K, 4, NH),
                                 wp1cm, wp2cat, Wa2, Wproj, Wf1, Wf2, g2b2))
    return jnp.concatenate(zs).reshape(B, N, DIM)
